# baseline (speedup 1.0000x reference)
import functools

import jax
import jax.numpy as jnp
from jax import lax
from jax.experimental import pallas as pl
from jax.experimental.pallas import tpu as pltpu
from jax.experimental.pallas import tpu_sc as plsc

_E = 64
_T = 16384
_IDX_N = 32768
_NC = 2
_NS = 16
_NW = _NC * _NS
_IPW = _IDX_N // _NW
_L = 16

_W = 128
_ROWS2 = _T * _E // _W
_GRID = 8
_BLK = _ROWS2 // _GRID


def _sc_hist_body(idx_hbm, out_hbm, idx_v, hist_v, out_v):
    wid = lax.axis_index("s") * _NC + lax.axis_index("c")
    pltpu.sync_copy(idx_hbm.at[pl.ds(wid * _IPW, _IPW)], idx_v)

    zero16 = jnp.zeros((_L,), jnp.float32)
    for i in range(_L * _E // _L):
        hist_v[pl.ds(i * _L, _L)] = zero16

    lane_base = lax.broadcasted_iota(jnp.int32, (_L,), 0) * _E
    ones = jnp.ones((_L,), jnp.float32)
    for i in range(_IPW // _L):
        v = idx_v[pl.ds(i * _L, _L)]
        plsc.addupdate_scatter(hist_v, [lane_base + v], ones)

    for j in range(_E // _L):
        acc = zero16
        for r in range(_L):
            acc = acc + hist_v[pl.ds(r * _E + j * _L, _L)]
        out_v[pl.ds(j * _L, _L)] = acc

    pltpu.sync_copy(out_v, out_hbm.at[pl.ds(wid * _E, _E)])


@functools.cache
def _sc_hist():
    return functools.partial(
        pl.kernel,
        mesh=plsc.VectorSubcoreMesh(core_axis_name="c", subcore_axis_name="s"),
        out_type=jax.ShapeDtypeStruct((_NW * _E,), jnp.float32),
        scratch_types=[
            pltpu.VMEM((_IPW,), jnp.int32),
            pltpu.VMEM((_L * _E,), jnp.float32),
            pltpu.VMEM((_E,), jnp.float32),
        ],
        compiler_params=pltpu.CompilerParams(needs_layout_passes=False),
    )(_sc_hist_body)


def _tc_softmax_body(x_ref, b_ref, acc_ref):
    step = pl.program_id(0)

    @pl.when(step == 0)
    def _init():
        acc_ref[...] = jnp.zeros_like(acc_ref)

    e = jnp.exp(x_ref[...])
    s = jnp.dot(e, b_ref[...], preferred_element_type=jnp.float32)
    p = e / s
    acc_ref[...] += jnp.sum(p, axis=0, keepdims=True)


def _tc_softmax(logits2, bdiag):
    return pl.pallas_call(
        _tc_softmax_body,
        grid=(_GRID,),
        in_specs=[
            pl.BlockSpec((_BLK, _W), lambda i: (i, 0)),
            pl.BlockSpec((_W, _W), lambda i: (0, 0)),
        ],
        out_specs=pl.BlockSpec((1, _W), lambda i: (0, 0)),
        out_shape=jax.ShapeDtypeStruct((1, _W), jnp.float32),
    )(logits2, bdiag)


def _tc_combine_body(hist_ref, ps_ref, loss_ref):
    counts = jnp.sum(hist_ref[...], axis=0, keepdims=True)
    ps = ps_ref[...]
    fold = ps[:, :_E] + ps[:, _E:]
    loss_ref[0, 0] = jnp.sum(fold * counts) * (_E / (_T * _T))


def _tc_combine(hist, probsum):
    return pl.pallas_call(
        _tc_combine_body,
        in_specs=[
            pl.BlockSpec((_NW, _E), lambda: (0, 0)),
            pl.BlockSpec((1, _W), lambda: (0, 0)),
        ],
        out_specs=pl.BlockSpec(memory_space=pltpu.SMEM),
        out_shape=jax.ShapeDtypeStruct((1, 1), jnp.float32),
    )(hist, probsum)


def kernel(router_logits, expert_indices):
    logits2 = router_logits.reshape(_ROWS2, _W)
    idx = expert_indices.astype(jnp.int32).reshape(_IDX_N)
    bdiag = jnp.kron(jnp.eye(2, dtype=jnp.float32), jnp.ones((_E, _E), jnp.float32))
    hist = _sc_hist()(idx).reshape(_NW, _E)
    probsum = _tc_softmax(logits2, bdiag)
    return _tc_combine(hist, probsum)[0, 0]


# speedup vs baseline: 1.0530x; 1.0530x over previous
import functools

import jax
import jax.numpy as jnp
from jax import lax
from jax.experimental import pallas as pl
from jax.experimental.pallas import tpu as pltpu
from jax.experimental.pallas import tpu_sc as plsc

_E = 64
_T = 16384
_IDX_N = 32768
_NC = 1
_NS = 16
_NW = _NC * _NS
_IPW = _IDX_N // _NW
_L = 16

_W = 128
_ROWS2 = _T * _E // _W
_GRID = 8
_BLK = _ROWS2 // _GRID


def _sc_hist_body(idx_hbm, out_hbm, idx_v, hist_v, out_v):
    wid = lax.axis_index("s") * _NC + lax.axis_index("c")
    pltpu.sync_copy(idx_hbm.at[pl.ds(wid * _IPW, _IPW)], idx_v)

    zero16 = jnp.zeros((_L,), jnp.float32)
    for i in range(_L * _E // _L):
        hist_v[pl.ds(i * _L, _L)] = zero16

    lane_base = lax.broadcasted_iota(jnp.int32, (_L,), 0) * _E
    ones = jnp.ones((_L,), jnp.float32)
    for i in range(_IPW // _L):
        v = idx_v[pl.ds(i * _L, _L)]
        plsc.addupdate_scatter(hist_v, [lane_base + v], ones)

    for j in range(_E // _L):
        acc = zero16
        for r in range(_L):
            acc = acc + hist_v[pl.ds(r * _E + j * _L, _L)]
        out_v[pl.ds(j * _L, _L)] = acc

    pltpu.sync_copy(out_v, out_hbm.at[pl.ds(wid * _E, _E)])


@functools.cache
def _sc_hist():
    return functools.partial(
        pl.kernel,
        mesh=plsc.VectorSubcoreMesh(
            core_axis_name="c", subcore_axis_name="s", num_cores=_NC),
        out_type=jax.ShapeDtypeStruct((_NW * _E,), jnp.float32),
        scratch_types=[
            pltpu.VMEM((_IPW,), jnp.int32),
            pltpu.VMEM((_L * _E,), jnp.float32),
            pltpu.VMEM((_E,), jnp.float32),
        ],
        compiler_params=pltpu.CompilerParams(needs_layout_passes=False),
    )(_sc_hist_body)


def _tc_body(x_ref, b_ref, hist_ref, loss_ref, acc_ref):
    step = pl.program_id(0)

    @pl.when(step == 0)
    def _init():
        acc_ref[...] = jnp.zeros_like(acc_ref)

    e = jnp.exp(x_ref[...])
    s = jnp.dot(e, b_ref[...], preferred_element_type=jnp.float32)
    acc_ref[...] += jnp.sum(e / s, axis=0, keepdims=True)

    @pl.when(step == _GRID - 1)
    def _fin():
        counts = jnp.sum(hist_ref[...], axis=0, keepdims=True)
        ps = acc_ref[...]
        fold = ps[:, :_E] + ps[:, _E:]
        loss_ref[0, 0] = jnp.sum(fold * counts) * (_E / (_T * _T))


def _tc_call(logits2, bdiag, hist):
    return pl.pallas_call(
        _tc_body,
        grid=(_GRID,),
        in_specs=[
            pl.BlockSpec((_BLK, _W), lambda i: (i, 0)),
            pl.BlockSpec((_W, _W), lambda i: (0, 0)),
            pl.BlockSpec((_NW, _E), lambda i: (0, 0)),
        ],
        out_specs=pl.BlockSpec(memory_space=pltpu.SMEM),
        out_shape=jax.ShapeDtypeStruct((1, 1), jnp.float32),
        scratch_shapes=[pltpu.VMEM((1, _W), jnp.float32)],
    )(logits2, bdiag, hist)


def kernel(router_logits, expert_indices):
    logits2 = router_logits.reshape(_ROWS2, _W)
    idx = expert_indices.astype(jnp.int32).reshape(_IDX_N)
    bdiag = jnp.kron(jnp.eye(2, dtype=jnp.float32), jnp.ones((_E, _E), jnp.float32))
    hist = _sc_hist()(idx).reshape(_NW, _E)
    return jnp.reshape(_tc_call(logits2, bdiag, hist), ())


# speedup vs baseline: 1.4280x; 1.3561x over previous
import jax
import jax.numpy as jnp
from jax import lax
from jax.experimental import pallas as pl
from jax.experimental.pallas import tpu as pltpu

_E = 64
_T = 16384
_IDX_N = 32768
_W = 128
_ROWS2 = _T * _E // _W
_GRID = 8
_BLK = _ROWS2 // _GRID
_IB = _IDX_N // _W // _GRID


def _body(x_ref, b_ref, idx_ref, loss_ref, acc_ref, hacc_ref):
    step = pl.program_id(0)

    @pl.when(step == 0)
    def _init():
        acc_ref[...] = jnp.zeros_like(acc_ref)
        hacc_ref[...] = jnp.zeros_like(hacc_ref)

    e = jnp.exp(x_ref[...])
    s = jnp.dot(e, b_ref[...], preferred_element_type=jnp.float32)
    acc_ref[...] += jnp.sum(e / s, axis=0, keepdims=True)

    x_i = idx_ref[...]
    e3 = lax.broadcasted_iota(jnp.int32, (_E, _IB, _W), 0)
    cmp = (x_i == e3).astype(jnp.float32)
    hacc_ref[...] += jnp.sum(cmp, axis=1)

    @pl.when(step == _GRID - 1)
    def _fin():
        counts = jnp.sum(hacc_ref[...], axis=1, keepdims=True)
        ps = acc_ref[...]
        fold = ps[:, :_E] + ps[:, _E:]
        loss = jnp.dot(fold, counts, preferred_element_type=jnp.float32)
        loss_ref[0, 0] = loss[0, 0] * (_E / (_T * _T))


def _call(logits2, bdiag, idx3, interpret=False):
    return pl.pallas_call(
        _body,
        grid=(_GRID,),
        in_specs=[
            pl.BlockSpec((_BLK, _W), lambda i: (i, 0)),
            pl.BlockSpec((_W, _W), lambda i: (0, 0)),
            pl.BlockSpec((1, _IB, _W), lambda i: (i, 0, 0)),
        ],
        out_specs=pl.BlockSpec(memory_space=pltpu.SMEM),
        out_shape=jax.ShapeDtypeStruct((1, 1), jnp.float32),
        scratch_shapes=[
            pltpu.VMEM((1, _W), jnp.float32),
            pltpu.VMEM((_E, _W), jnp.float32),
        ],
        interpret=interpret,
    )(logits2, bdiag, idx3)


def kernel(router_logits, expert_indices, interpret=False):
    logits2 = router_logits.reshape(_ROWS2, _W)
    idx3 = expert_indices.astype(jnp.int32).reshape(_GRID, _IB, _W)
    bdiag = jnp.kron(jnp.eye(2, dtype=jnp.float32),
                     jnp.ones((_E, _E), jnp.float32))
    return jnp.reshape(_call(logits2, bdiag, idx3, interpret=interpret), ())


# speedup vs baseline: 6.0681x; 4.2495x over previous
import jax
import jax.numpy as jnp
from jax import lax
from jax.experimental import pallas as pl
from jax.experimental.pallas import tpu as pltpu

_E = 64
_B = 4
_T = 16384
_TOK = 4096
_GRID = 8
_TB = _TOK // _GRID
_R = _B * _E
_K2 = 8


def _body(x_ref, idx_ref, loss_ref, acc_ref, hacc_ref):
    step = pl.program_id(0)

    @pl.when(step == 0)
    def _init():
        acc_ref[...] = jnp.zeros_like(acc_ref)
        hacc_ref[...] = jnp.zeros_like(hacc_ref)

    ex = jnp.exp(x_ref[...])
    for b in range(_B):
        eb = ex[b * _E:(b + 1) * _E, :]
        rb = 1.0 / jnp.sum(eb, axis=0, keepdims=True)
        pb = eb * rb
        acc_ref[...] += (pb[:, 0:128] + pb[:, 128:256]
                         + pb[:, 256:384] + pb[:, 384:512])

    xi = idx_ref[...]
    e3 = lax.broadcasted_iota(jnp.int32, (_E, _K2, _TB), 0)
    cmp = (xi[None, :, :] == e3).astype(jnp.float32)
    hacc_ref[...] += (cmp[:, :, 0:128] + cmp[:, :, 128:256]
                      + cmp[:, :, 256:384] + cmp[:, :, 384:512])

    @pl.when(step == _GRID - 1)
    def _fin():
        probcol = jnp.sum(acc_ref[...], axis=1, keepdims=True)
        hcol = jnp.sum(jnp.sum(hacc_ref[...], axis=1), axis=1,
                       keepdims=True)
        loss_ref[0, 0] = jnp.sum(probcol * hcol) * (_E / (_T * _T))


def kernel(router_logits, expert_indices):
    x_t = router_logits.transpose(0, 2, 1).reshape(_R, _TOK)
    idx_t = expert_indices.astype(jnp.int32).transpose(0, 2, 1).reshape(_K2, _TOK)
    out = pl.pallas_call(
        _body,
        grid=(_GRID,),
        in_specs=[
            pl.BlockSpec((_R, _TB), lambda i: (0, i)),
            pl.BlockSpec((_K2, _TB), lambda i: (0, i)),
        ],
        out_specs=pl.BlockSpec(memory_space=pltpu.SMEM),
        out_shape=jax.ShapeDtypeStruct((1, 1), jnp.float32),
        scratch_shapes=[
            pltpu.VMEM((_E, 128), jnp.float32),
            pltpu.VMEM((_E, _K2, 128), jnp.float32),
        ],
    )(x_t, idx_t)
    return jnp.reshape(out, ())


# speedup vs baseline: 7.5188x; 1.2391x over previous
import jax
import jax.numpy as jnp
from jax import lax
from jax.experimental import pallas as pl
from jax.experimental.pallas import tpu as pltpu

_E = 64
_B = 4
_T = 16384
_TOK = 4096
_GRID = 4
_TB = _TOK // _GRID
_R = _B * _E
_K2 = 8


def _body(x_ref, idx_ref, loss_ref, acc_ref, hacc_ref):
    step = pl.program_id(0)

    @pl.when(step == 0)
    def _init():
        acc_ref[...] = jnp.zeros_like(acc_ref)
        hacc_ref[...] = jnp.zeros_like(hacc_ref)

    ex = jnp.exp(x_ref[...])
    for b in range(_B):
        eb = ex[b * _E:(b + 1) * _E, :]
        rb = 1.0 / jnp.sum(eb, axis=0, keepdims=True)
        pb = eb * rb
        fold = pb[:, 0:128]
        for k in range(1, _TB // 128):
            fold = fold + pb[:, k * 128:(k + 1) * 128]
        acc_ref[...] += fold

    xi = idx_ref[...]
    e3 = lax.broadcasted_iota(jnp.int32, (_E, _K2, _TB), 0)
    cmp = (xi[None, :, :] == e3).astype(jnp.float32)
    hfold = cmp[:, :, 0:128]
    for k in range(1, _TB // 128):
        hfold = hfold + cmp[:, :, k * 128:(k + 1) * 128]
    hacc_ref[...] += hfold

    @pl.when(step == _GRID - 1)
    def _fin():
        probcol = jnp.sum(acc_ref[...], axis=1, keepdims=True)
        hcol = jnp.sum(jnp.sum(hacc_ref[...], axis=1), axis=1,
                       keepdims=True)
        loss_ref[0, 0] = jnp.sum(probcol * hcol) * (_E / (_T * _T))


def kernel(router_logits, expert_indices):
    x_t = router_logits.transpose(0, 2, 1).reshape(_R, _TOK)
    idx_t = expert_indices.astype(jnp.int32).transpose(0, 2, 1).reshape(_K2, _TOK)
    out = pl.pallas_call(
        _body,
        grid=(_GRID,),
        in_specs=[
            pl.BlockSpec((_R, _TB), lambda i: (0, i)),
            pl.BlockSpec((_K2, _TB), lambda i: (0, i)),
        ],
        out_specs=pl.BlockSpec(memory_space=pltpu.SMEM),
        out_shape=jax.ShapeDtypeStruct((1, 1), jnp.float32),
        scratch_shapes=[
            pltpu.VMEM((_E, 128), jnp.float32),
            pltpu.VMEM((_E, _K2, 128), jnp.float32),
        ],
    )(x_t, idx_t)
    return jnp.reshape(out, ())
